# baseline (speedup 1.0000x reference)
import jax
import jax.numpy as jnp
from jax.experimental import pallas as pl
from jax.experimental.pallas import tpu as pltpu

K, B, N, D = 6, 64, 100000, 128
TN = 12800
NB = pl.cdiv(N, TN)
RD = 4
T = K * NB
REM = N - (NB - 1) * TN
OFF = TN - REM


def _sim_body(pf_ref, mem_ref, out_ref, f16_ref, in_bufs, rsems):
    k = pl.program_id(0)
    n = pl.program_id(1)
    i = k * NB + n

    @pl.when(n == 0)
    def _():
        f = pf_ref[0]
        norm = jnp.sqrt(jnp.sum(f * f, axis=1, keepdims=True))
        f16_ref[...] = (f / jnp.maximum(norm, 1e-12)).astype(jnp.bfloat16)

    def read_copy(jk, jn, slot):
        off = jnp.minimum(jn * TN, N - TN)
        return pltpu.make_async_copy(
            mem_ref.at[jk, pl.ds(off, TN), :],
            in_bufs.at[slot],
            rsems.at[slot],
        )

    @pl.when(i == 0)
    def _():
        for d in range(RD):
            read_copy(0, d, d).start()

    @pl.when((i > 0) & (i + RD - 1 < T))
    def _():
        j = i + RD - 1
        read_copy(j // NB, j % NB, j % RD).start()

    read_copy(k, n, i % RD).wait()
    m = in_bufs[i % RD]

    @pl.when(n < NB - 1)
    def _():
        out_ref[0] = jax.lax.dot_general(
            f16_ref[...], m.astype(jnp.bfloat16),
            (((1,), (1,)), ((), ())), preferred_element_type=jnp.float32,
        )

    @pl.when(n == NB - 1)
    def _():
        res = jax.lax.dot_general(
            f16_ref[...], m[OFF:].astype(jnp.bfloat16),
            (((1,), (1,)), ((), ())), preferred_element_type=jnp.float32,
        )
        out_ref[0, :, :REM] = res


def kernel(part_features, memory):
    return pl.pallas_call(
        _sim_body,
        grid=(K, NB),
        in_specs=[
            pl.BlockSpec((1, B, D), lambda k, n: (k, 0, 0)),
            pl.BlockSpec(memory_space=pl.ANY),
        ],
        out_specs=pl.BlockSpec((1, B, TN), lambda k, n: (k, 0, n)),
        out_shape=jax.ShapeDtypeStruct((K, B, N), jnp.float32),
        scratch_shapes=[
            pltpu.VMEM((B, D), jnp.bfloat16),
            pltpu.VMEM((RD, TN, D), jnp.float32),
            pltpu.SemaphoreType.DMA((RD,)),
        ],
        compiler_params=pltpu.CompilerParams(
            dimension_semantics=("arbitrary", "arbitrary"),
        ),
    )(part_features, memory)


# speedup vs baseline: 1.0198x; 1.0198x over previous
import jax
import jax.numpy as jnp
from jax.experimental import pallas as pl
from jax.experimental.pallas import tpu as pltpu

K, B, N, D = 6, 64, 100000, 128
TN = 33408


def _sim_body(pf_ref, mem_ref, out_ref, f16_ref):
    n = pl.program_id(1)

    @pl.when(n == 0)
    def _():
        f = pf_ref[0]
        norm = jnp.sqrt(jnp.sum(f * f, axis=1, keepdims=True))
        f16_ref[...] = (f / jnp.maximum(norm, 1e-12)).astype(jnp.bfloat16)

    m = mem_ref[0].astype(jnp.bfloat16)
    out_ref[0] = jax.lax.dot_general(
        f16_ref[...], m, (((1,), (1,)), ((), ())),
        preferred_element_type=jnp.float32,
    )


def kernel(part_features, memory):
    nb = pl.cdiv(N, TN)
    return pl.pallas_call(
        _sim_body,
        grid=(K, nb),
        in_specs=[
            pl.BlockSpec((1, B, D), lambda k, n: (k, 0, 0)),
            pl.BlockSpec((1, TN, D), lambda k, n: (k, n, 0)),
        ],
        out_specs=pl.BlockSpec((1, B, TN), lambda k, n: (k, 0, n)),
        out_shape=jax.ShapeDtypeStruct((K, B, N), jnp.float32),
        scratch_shapes=[pltpu.VMEM((B, D), jnp.bfloat16)],
        compiler_params=pltpu.CompilerParams(
            dimension_semantics=("parallel", "arbitrary"),
        ),
    )(part_features, memory)
